# Initial kernel scaffold; baseline (speedup 1.0000x reference)
import jax
import jax.numpy as jnp
from jax.experimental import pallas as pl


def kernel(feature, edge_index, adj_values, conv_w, conv_b, W1, b1, W2, b2):
    raise NotImplementedError("write your pallas kernel here")


# speedup vs baseline: 9.4931x; 9.4931x over previous
import functools

import jax
import jax.numpy as jnp
from jax import lax
from jax.experimental import pallas as pl
from jax.experimental.pallas import tpu as pltpu
from jax.experimental.pallas import tpu_sc as plsc

N_NODES = 50000
N_EDGES = 800000
IN_DIM = 83
HIDDEN = 1024
NUM_CLASSES = 25

NC = 2
NS = 16
LANES = 16

CH = 128
NCHUNK = 392
WSUP = 56
NSUP = NCHUNK // WSUP
ESUB = NCHUNK * CH
EPAD = NS * ESUB
NPAD = 50048
RPS = NPAD // NS

BLK = 2000
GRID = N_NODES // BLK

D1 = 96
G1 = D1 // LANES
D2 = 32
G2 = D2 // LANES


@functools.lru_cache(maxsize=None)
def _make_spmm(n_groups):
  gpc = n_groups // NC
  mesh = plsc.VectorSubcoreMesh(
      core_axis_name="c", subcore_axis_name="s", num_cores=NC, num_subcores=NS)

  @functools.partial(
      pl.kernel,
      out_type=jax.ShapeDtypeStruct((n_groups, NPAD, LANES), jnp.float32),
      mesh=mesh,
      scratch_types=[
          pltpu.VMEM((WSUP, CH), jnp.int32),
          pltpu.VMEM((WSUP, CH), jnp.int32),
          pltpu.VMEM((WSUP, CH), jnp.float32),
          pltpu.VMEM((CH, LANES), jnp.float32),
          pltpu.VMEM((RPS, LANES), jnp.float32),
          pltpu.VMEM((LANES,), jnp.float32),
          pltpu.VMEM_SHARED((NPAD, LANES), jnp.float32),
          pltpu.SemaphoreType.DMA,
      ],
      compiler_params=pltpu.CompilerParams(use_tc_tiling_on_sc=False),
  )
  def spmm(*args):
    tables = args[:n_groups]
    colsr, rowsr, adjr, binit, out = args[n_groups:n_groups + 5]
    cols_v, rows_v, adj_v, gbuf, obuf, bbuf, acc, sem = args[n_groups + 5:]
    c = lax.axis_index("c")
    s = lax.axis_index("s")

    for g in range(n_groups):
      @pl.when(c == g // gpc)
      def _(g=g):
        tbl = tables[g]
        pltpu.sync_copy(binit.at[g], bbuf)
        bv = bbuf[...]

        def _init(i, _):
          obuf[i] = bv
          return 0

        lax.fori_loop(0, RPS, _init, 0)
        pltpu.sync_copy(obuf, acc.at[pl.ds(s * RPS, RPS)])
        plsc.subcore_barrier()

        for w in range(NSUP):
          pltpu.sync_copy(colsr.at[s, pl.ds(w * WSUP, WSUP)], cols_v)
          pltpu.sync_copy(rowsr.at[s, pl.ds(w * WSUP, WSUP)], rows_v)
          pltpu.sync_copy(adjr.at[s, pl.ds(w * WSUP, WSUP)], adj_v)

          def _chunk(j, _):
            pltpu.async_copy(tbl.at[cols_v.at[j]], gbuf, sem).wait()

            def _edge16(k, _):
              adjv = adj_v[j, pl.ds(k * LANES, LANES)]
              base = k * LANES
              for l in range(LANES):
                gbuf[base + l] = gbuf[base + l] * adjv[l]
              return 0

            lax.fori_loop(0, CH // LANES, _edge16, 0)
            pltpu.sync_copy(gbuf, acc.at[rows_v.at[j]], add=True)
            return 0

          lax.fori_loop(0, WSUP, _chunk, 0)

        plsc.subcore_barrier()
        pltpu.sync_copy(acc.at[pl.ds(s * RPS, RPS)], obuf)
        pltpu.sync_copy(obuf, out.at[g, pl.ds(s * RPS, RPS)])

  return spmm


def _tc1_body(f_ref, c_ref, b_ref, *outs):
  x = jnp.dot(f_ref[...], c_ref[...], preferred_element_type=jnp.float32)
  x = jnp.maximum(x + b_ref[...], 0.0)
  for g in range(G1):
    outs[g][...] = x[:, g * LANES:(g + 1) * LANES]


def _tc2_body(*refs):
  s_refs = refs[:G1]
  w1_ref, b1_ref, w2_ref, o0, o1 = refs[G1:]
  s = jnp.concatenate([r[...][0] for r in s_refs], axis=1)
  h = jnp.dot(s, w1_ref[...], preferred_element_type=jnp.float32)
  h = jnp.maximum(h + b1_ref[...], 0.0)
  o = jnp.dot(h, w2_ref[...], preferred_element_type=jnp.float32)
  o0[...] = o[:, :LANES]
  o1[...] = o[:, LANES:]


def kernel(feature, edge_index, adj_values, conv_w, conv_b, W1, b1, W2, b2):
  f32 = jnp.float32
  rows = edge_index[0]
  cols = edge_index[1]

  pad = EPAD - N_EDGES
  colsr = jnp.pad(cols, (0, pad)).reshape(NS, NCHUNK, CH)
  rowsr = jnp.pad(rows, (0, pad)).reshape(NS, NCHUNK, CH)
  adjr = jnp.pad(adj_values, (0, pad)).reshape(NS, NCHUNK, CH)

  wsum = jnp.sum(conv_w[:, 0, :], axis=0)
  bsum = jnp.sum(conv_b)
  cm = jnp.zeros((IN_DIM, IN_DIM), f32)
  for t in range(5):
    cm = cm + wsum[t] * jnp.eye(IN_DIM, k=2 - t, dtype=f32)
  cpad = jnp.zeros((D1, D1), f32).at[:IN_DIM, :IN_DIM].set(cm)
  bvec = jnp.zeros((1, D1), f32).at[0, :IN_DIM].set(bsum)
  fpad = jnp.pad(feature, ((0, 0), (0, D1 - IN_DIM)))

  tables1 = pl.pallas_call(
      _tc1_body,
      grid=(GRID,),
      in_specs=[
          pl.BlockSpec((BLK, D1), lambda i: (i, 0)),
          pl.BlockSpec((D1, D1), lambda i: (0, 0)),
          pl.BlockSpec((1, D1), lambda i: (0, 0)),
      ],
      out_specs=[pl.BlockSpec((BLK, LANES), lambda i: (i, 0))] * G1,
      out_shape=[jax.ShapeDtypeStruct((N_NODES, LANES), f32)] * G1,
  )(fpad, cpad, bvec)

  binit1 = jnp.zeros((G1, LANES), f32)
  s1 = _make_spmm(G1)(*tables1, colsr, rowsr, adjr, binit1)

  w1p = jnp.zeros((D1, HIDDEN), f32).at[:IN_DIM].set(W1)
  b1r = b1.reshape(1, HIDDEN)
  w2p = jnp.pad(W2, ((0, 0), (0, D2 - NUM_CLASSES)))
  s_spec = [
      pl.BlockSpec((1, BLK, LANES), functools.partial(lambda i, g: (g, i, 0), g=g))
      for g in range(G1)
  ]
  tables2 = pl.pallas_call(
      _tc2_body,
      grid=(GRID,),
      in_specs=s_spec + [
          pl.BlockSpec((D1, HIDDEN), lambda i: (0, 0)),
          pl.BlockSpec((1, HIDDEN), lambda i: (0, 0)),
          pl.BlockSpec((HIDDEN, D2), lambda i: (0, 0)),
      ],
      out_specs=[pl.BlockSpec((BLK, LANES), lambda i: (i, 0))] * G2,
      out_shape=[jax.ShapeDtypeStruct((N_NODES, LANES), f32)] * G2,
  )(*([s1] * G1), w1p, b1r, w2p)

  binit2 = jnp.pad(b2, (0, D2 - NUM_CLASSES)).reshape(G2, LANES)
  o2 = _make_spmm(G2)(*tables2, colsr, rowsr, adjr, binit2)

  logits = o2[:, :N_NODES, :].transpose(1, 0, 2).reshape(N_NODES, G2 * LANES)
  return logits[:, :NUM_CLASSES]


# speedup vs baseline: 16.8507x; 1.7750x over previous
import functools

import jax
import jax.numpy as jnp
from jax import lax
from jax.experimental import pallas as pl
from jax.experimental.pallas import tpu as pltpu
from jax.experimental.pallas import tpu_sc as plsc

N_NODES = 50000
N_EDGES = 800000
IN_DIM = 83
HIDDEN = 1024
NUM_CLASSES = 25

NC = 2
NS = 16
LANES = 16

G1 = 6
G2 = 2
D1 = G1 * LANES
D2 = G2 * LANES

CH = 128
NCHUNK = 392
KF = 7
WSUP = 28
NSUP = NCHUNK // WSUP
GSUP = WSUP // KF
EPAD = NS * NCHUNK * CH
EROWS = N_EDGES // CH
PROWS = EPAD // CH
NPAD = 50048
RPS = NPAD // NS

BLK = 2000
GRID = N_NODES // BLK


@functools.lru_cache(maxsize=None)
def _make_spmm(n_groups):
  gpc = n_groups // NC
  mesh = plsc.VectorSubcoreMesh(
      core_axis_name="c", subcore_axis_name="s", num_cores=NC, num_subcores=NS)

  @functools.partial(
      pl.kernel,
      out_type=jax.ShapeDtypeStruct((n_groups, NPAD, LANES), jnp.float32),
      mesh=mesh,
      scratch_types=[
          pltpu.VMEM((WSUP, CH), jnp.int32),
          pltpu.VMEM((WSUP, CH), jnp.int32),
          pltpu.VMEM((WSUP, CH), jnp.float32),
          pltpu.VMEM((KF, CH, LANES), jnp.float32),
          pltpu.VMEM((RPS, LANES), jnp.float32),
          pltpu.VMEM((LANES,), jnp.float32),
          pltpu.VMEM_SHARED((NPAD, LANES), jnp.float32),
          pltpu.SemaphoreType.DMA,
          pltpu.SemaphoreType.DMA,
      ],
      compiler_params=pltpu.CompilerParams(use_tc_tiling_on_sc=False),
  )
  def spmm(*args):
    tables = args[:n_groups]
    colsr, rowsr, adjr, binit, out = args[n_groups:n_groups + 5]
    (cols_v, rows_v, adj_v, gbuf, obuf, bbuf, acc, gsem, ssem) = \
        args[n_groups + 5:]
    c = lax.axis_index("c")
    s = lax.axis_index("s")

    for g in range(n_groups):
      @pl.when(c == g // gpc)
      def _(g=g):
        tbl = tables[g]
        pltpu.sync_copy(binit.at[g], bbuf)
        bv = bbuf[...]

        def _init(i, _):
          obuf[i] = bv
          return 0

        lax.fori_loop(0, RPS, _init, 0)
        pltpu.sync_copy(obuf, acc.at[pl.ds(s * RPS, RPS)])
        plsc.subcore_barrier()

        def _sup(w, _):
          pltpu.sync_copy(colsr.at[s, pl.ds(w * WSUP, WSUP)], cols_v)
          pltpu.sync_copy(rowsr.at[s, pl.ds(w * WSUP, WSUP)], rows_v)
          pltpu.sync_copy(adjr.at[s, pl.ds(w * WSUP, WSUP)], adj_v)

          def _grp(g2, _):
            j0 = g2 * KF
            gd = [
                pltpu.async_copy(tbl.at[cols_v.at[j0 + b]], gbuf.at[b], gsem)
                for b in range(KF)
            ]
            sd = []
            for b in range(KF):
              gd[b].wait()

              def _edge16(k, _, b=b):
                adjv = adj_v[j0 + b, pl.ds(k * LANES, LANES)]
                base = k * LANES
                for l in range(LANES):
                  e = base + l
                  gbuf[b, e] = gbuf[b, e] * adjv[l]
                return 0

              lax.fori_loop(0, CH // LANES, _edge16, 0)
              sd.append(
                  pltpu.async_copy(
                      gbuf.at[b], acc.at[rows_v.at[j0 + b]], ssem, add=True))
            for d in sd:
              d.wait()
            return 0

          lax.fori_loop(0, GSUP, _grp, 0)
          return 0

        lax.fori_loop(0, NSUP, _sup, 0)

        plsc.subcore_barrier()
        pltpu.sync_copy(acc.at[pl.ds(s * RPS, RPS)], obuf)
        pltpu.sync_copy(obuf, out.at[g, pl.ds(s * RPS, RPS)])

  return spmm


def _pad_body(c_ref, r_ref, a_ref, co_ref, ro_ref, ao_ref):
  zi = jnp.zeros((PROWS - EROWS, CH), jnp.int32)
  co_ref[:EROWS] = c_ref[...]
  co_ref[EROWS:] = zi
  ro_ref[:EROWS] = r_ref[...]
  ro_ref[EROWS:] = zi
  ao_ref[:EROWS] = a_ref[...]
  ao_ref[EROWS:] = jnp.zeros((PROWS - EROWS, CH), jnp.float32)


def _tc1_body(f_ref, c_ref, b_ref, *outs):
  x = jnp.dot(f_ref[...], c_ref[...], preferred_element_type=jnp.float32)
  x = jnp.maximum(x + b_ref[...], 0.0)
  for g in range(G1):
    outs[g][...] = x[:, g * LANES:(g + 1) * LANES]


def _tc2_body(*refs):
  s_refs = refs[:G1]
  w1_ref, b1_ref, w2_ref, *outs = refs[G1:]
  s = jnp.concatenate([r[...][0] for r in s_refs], axis=1)[:, :IN_DIM]
  h = jnp.dot(s, w1_ref[...], preferred_element_type=jnp.float32)
  h = jnp.maximum(h + b1_ref[...], 0.0)
  o = jnp.dot(h, w2_ref[...], preferred_element_type=jnp.float32)
  o = jnp.concatenate(
      [o, jnp.zeros((BLK, D2 - NUM_CLASSES), jnp.float32)], axis=1)
  for g in range(G2):
    outs[g][...] = o[:, g * LANES:(g + 1) * LANES]


def _tc3_body(p_ref, out):
  p = p_ref[...]
  out[...] = jnp.concatenate([p[g] for g in range(G2)],
                             axis=1)[:, :NUM_CLASSES]


def kernel(feature, edge_index, adj_values, conv_w, conv_b, W1, b1, W2, b2):
  f32 = jnp.float32
  rows = edge_index[0]
  cols = edge_index[1]

  full_spec = pl.BlockSpec((EROWS, CH), lambda: (0, 0))
  pad_spec = pl.BlockSpec((PROWS, CH), lambda: (0, 0))
  colsp, rowsp, adjp = pl.pallas_call(
      _pad_body,
      grid=(),
      in_specs=[full_spec] * 3,
      out_specs=[pad_spec] * 3,
      out_shape=[
          jax.ShapeDtypeStruct((PROWS, CH), jnp.int32),
          jax.ShapeDtypeStruct((PROWS, CH), jnp.int32),
          jax.ShapeDtypeStruct((PROWS, CH), f32),
      ],
  )(cols.reshape(EROWS, CH), rows.reshape(EROWS, CH),
    adj_values.reshape(EROWS, CH))
  colsr = colsp.reshape(NS, NCHUNK, CH)
  rowsr = rowsp.reshape(NS, NCHUNK, CH)
  adjr = adjp.reshape(NS, NCHUNK, CH)

  wsum = jnp.sum(conv_w[:, 0, :], axis=0)
  bsum = jnp.sum(conv_b)
  cm = jnp.zeros((IN_DIM, D1), f32)
  for t in range(5):
    cm = cm.at[:, :IN_DIM].add(wsum[t] * jnp.eye(IN_DIM, k=2 - t, dtype=f32))
  bvec = jnp.zeros((1, D1), f32).at[0, :IN_DIM].set(bsum)

  tables1 = pl.pallas_call(
      _tc1_body,
      grid=(GRID,),
      in_specs=[
          pl.BlockSpec((BLK, IN_DIM), lambda i: (i, 0)),
          pl.BlockSpec((IN_DIM, D1), lambda i: (0, 0)),
          pl.BlockSpec((1, D1), lambda i: (0, 0)),
      ],
      out_specs=[pl.BlockSpec((BLK, LANES), lambda i: (i, 0))] * G1,
      out_shape=[jax.ShapeDtypeStruct((N_NODES, LANES), f32)] * G1,
  )(feature, cm, bvec)

  binit1 = jnp.zeros((G1, LANES), f32)
  s1 = _make_spmm(G1)(*tables1, colsr, rowsr, adjr, binit1)

  b1r = b1.reshape(1, HIDDEN)
  s_spec = [
      pl.BlockSpec((1, BLK, LANES),
                   functools.partial(lambda i, g: (g, i, 0), g=g))
      for g in range(G1)
  ]
  tables2 = pl.pallas_call(
      _tc2_body,
      grid=(GRID,),
      in_specs=s_spec + [
          pl.BlockSpec((IN_DIM, HIDDEN), lambda i: (0, 0)),
          pl.BlockSpec((1, HIDDEN), lambda i: (0, 0)),
          pl.BlockSpec((HIDDEN, NUM_CLASSES), lambda i: (0, 0)),
      ],
      out_specs=[pl.BlockSpec((BLK, LANES), lambda i: (i, 0))] * G2,
      out_shape=[jax.ShapeDtypeStruct((N_NODES, LANES), f32)] * G2,
  )(*([s1] * G1), W1, b1r, W2)

  b2p = jnp.zeros((D2,), f32).at[:NUM_CLASSES].set(b2)
  binit2 = b2p.reshape(G2, LANES)
  o2 = _make_spmm(G2)(*tables2, colsr, rowsr, adjr, binit2)

  logits = pl.pallas_call(
      _tc3_body,
      grid=(GRID,),
      in_specs=[pl.BlockSpec((G2, BLK, LANES), lambda i: (0, i, 0))],
      out_specs=pl.BlockSpec((BLK, NUM_CLASSES), lambda i: (i, 0)),
      out_shape=jax.ShapeDtypeStruct((N_NODES, NUM_CLASSES), f32),
  )(o2)

  return logits
